# Initial kernel scaffold; baseline (speedup 1.0000x reference)
import jax
import jax.numpy as jnp
from jax.experimental import pallas as pl


def kernel(node_feature, edge_matrix, edge_feature, W1, b1, W2, b2, W3, b3, ln_g, ln_b):
    raise NotImplementedError("write your pallas kernel here")


# speedup vs baseline: 3.6781x; 3.6781x over previous
import functools

import jax
import jax.numpy as jnp
from jax import lax
from jax.experimental import pallas as pl
from jax.experimental.pallas import tpu as pltpu
from jax.experimental.pallas import tpu_sc as plsc

N = 10000
E = 320000
D = 128
NC = 2
NS = 16
NW = NC * NS
EPT = E // NW
CHUNK = 80
NCHUNK = EPT // CHUNK
N_PAD = 10240
ROWS_PT = N_PAD // NS


def _sc_segment_sum(recv_idx, edge_feature, zeros):
    mesh = plsc.VectorSubcoreMesh(core_axis_name="c", subcore_axis_name="s")

    @functools.partial(
        pl.kernel,
        mesh=mesh,
        out_type=jax.ShapeDtypeStruct((NC, N_PAD, D), jnp.float32),
        scratch_types=[
            pltpu.VMEM((CHUNK,), jnp.int32),
            pltpu.VMEM((CHUNK, D), jnp.float32),
            pltpu.VMEM_SHARED((N_PAD, D), jnp.float32),
        ],
    )
    def k(idx_hbm, ef_hbm, zero_hbm, out_hbm, idx_v, rows_v, agg_sh):
        cid = lax.axis_index("c")
        sid = lax.axis_index("s")
        wid = sid * NC + cid
        pltpu.sync_copy(zero_hbm.at[pl.ds(sid * ROWS_PT, ROWS_PT)],
                        agg_sh.at[pl.ds(sid * ROWS_PT, ROWS_PT)])
        plsc.subcore_barrier()

        base = wid * EPT

        def body(i, carry):
            off = base + i * CHUNK
            pltpu.sync_copy(idx_hbm.at[pl.ds(off, CHUNK)], idx_v)
            pltpu.sync_copy(ef_hbm.at[pl.ds(off, CHUNK)], rows_v)
            pltpu.sync_copy(rows_v, agg_sh.at[idx_v], add=True)
            return carry

        lax.fori_loop(0, NCHUNK, body, 0)
        plsc.subcore_barrier()
        pltpu.sync_copy(agg_sh.at[pl.ds(sid * ROWS_PT, ROWS_PT)],
                        out_hbm.at[cid, pl.ds(sid * ROWS_PT, ROWS_PT)])

    return k(recv_idx, edge_feature, zeros)


BN = 1000


def _tc_mlp(node, parts, W1, b1, W2, b2, W3, b3, ln_g, ln_b):
    def body(node_ref, p_ref, w1_ref, b1_ref, w2_ref, b2_ref, w3_ref, b3_ref,
             g_ref, beta_ref, out_ref):
        x = node_ref[...]
        agg = p_ref[0] + p_ref[1]
        h = jnp.dot(x, w1_ref[:D, :], preferred_element_type=jnp.float32)
        h = h + jnp.dot(agg, w1_ref[D:, :], preferred_element_type=jnp.float32)
        h = jax.nn.relu(h + b1_ref[...])
        h = jax.nn.relu(
            jnp.dot(h, w2_ref[...], preferred_element_type=jnp.float32) + b2_ref[...])
        h = jnp.dot(h, w3_ref[...], preferred_element_type=jnp.float32) + b3_ref[...]
        mu = jnp.mean(h, axis=-1, keepdims=True)
        var = jnp.mean((h - mu) * (h - mu), axis=-1, keepdims=True)
        h = (h - mu) * lax.rsqrt(var + 1e-5) * g_ref[...] + beta_ref[...]
        out_ref[...] = h + x

    vec = lambda: pl.BlockSpec((1, D), lambda i: (0, 0))
    return pl.pallas_call(
        body,
        grid=(N // BN,),
        in_specs=[
            pl.BlockSpec((BN, D), lambda i: (i, 0)),
            pl.BlockSpec((NC, BN, D), lambda i: (0, i, 0)),
            pl.BlockSpec((2 * D, D), lambda i: (0, 0)),
            vec(),
            pl.BlockSpec((D, D), lambda i: (0, 0)),
            vec(),
            pl.BlockSpec((D, D), lambda i: (0, 0)),
            vec(),
            vec(),
            vec(),
        ],
        out_specs=pl.BlockSpec((BN, D), lambda i: (i, 0)),
        out_shape=jax.ShapeDtypeStruct((N, D), jnp.float32),
    )(node, parts, W1, b1.reshape(1, D), W2, b2.reshape(1, D), W3,
      b3.reshape(1, D), ln_g.reshape(1, D), ln_b.reshape(1, D))


def kernel(node_feature, edge_matrix, edge_feature, W1, b1, W2, b2, W3, b3,
           ln_g, ln_b):
    recv_idx = edge_matrix[1]
    zeros = jnp.zeros((N_PAD, D), dtype=jnp.float32)
    parts = _sc_segment_sum(recv_idx, edge_feature, zeros)
    return _tc_mlp(node_feature, parts, W1, b1, W2, b2, W3, b3, ln_g, ln_b)


# speedup vs baseline: 6.7243x; 1.8282x over previous
import functools

import jax
import jax.numpy as jnp
from jax import lax
from jax.experimental import pallas as pl
from jax.experimental.pallas import tpu as pltpu
from jax.experimental.pallas import tpu_sc as plsc

N = 10000
E = 320000
D = 128
NC = 2
NS = 16
NW = NC * NS
EPT = E // NW
SUB = 80
LC = 80
SPL = LC // SUB
NL = EPT // LC
NCHUNK = EPT // SUB
N_PAD = 10240
ROWS_PT = N_PAD // NS


def _sc_segment_sum(recv_idx3d, edge_feature, zeros):
    mesh = plsc.VectorSubcoreMesh(core_axis_name="c", subcore_axis_name="s")

    @functools.partial(
        pl.kernel,
        mesh=mesh,
        out_type=jax.ShapeDtypeStruct((NC, N_PAD, D), jnp.float32),
        scratch_types=[
            pltpu.VMEM((NCHUNK, SUB), jnp.int32),
            pltpu.VMEM((LC, D), jnp.float32),
            pltpu.VMEM((LC, D), jnp.float32),
            pltpu.VMEM_SHARED((N_PAD, D), jnp.float32),
            pltpu.SemaphoreType.DMA,
            pltpu.SemaphoreType.DMA,
        ],
    )
    def k(idx_hbm, ef_hbm, zero_hbm, out_hbm, idx_v, buf0, buf1, agg_sh,
          sem0, sem1):
        cid = lax.axis_index("c")
        sid = lax.axis_index("s")
        wid = sid * NC + cid
        pltpu.sync_copy(zero_hbm.at[pl.ds(sid * ROWS_PT, ROWS_PT)],
                        agg_sh.at[pl.ds(sid * ROWS_PT, ROWS_PT)])
        pltpu.sync_copy(idx_hbm.at[wid], idx_v)
        plsc.subcore_barrier()

        base = wid * EPT

        def load(i, buf, sem):
            return pltpu.async_copy(ef_hbm.at[pl.ds(base + i * LC, LC)], buf, sem)

        def scatter(i, buf):
            for j in range(SPL):
                pltpu.sync_copy(buf.at[pl.ds(j * SUB, SUB)],
                                agg_sh.at[idx_v.at[i * SPL + j]], add=True)

        load(0, buf0, sem0)

        def body(kk, carry):
            i0 = 2 * kk
            load(i0 + 1, buf1, sem1)
            pltpu.make_async_copy(ef_hbm.at[pl.ds(0, LC)], buf0, sem0).wait()
            scatter(i0, buf0)
            load(i0 + 2, buf0, sem0)
            pltpu.make_async_copy(ef_hbm.at[pl.ds(0, LC)], buf1, sem1).wait()
            scatter(i0 + 1, buf1)
            return carry

        lax.fori_loop(0, (NL - 1) // 2, body, 0)
        pltpu.make_async_copy(ef_hbm.at[pl.ds(0, LC)], buf0, sem0).wait()
        scatter(NL - 1, buf0)

        plsc.subcore_barrier()
        pltpu.sync_copy(agg_sh.at[pl.ds(sid * ROWS_PT, ROWS_PT)],
                        out_hbm.at[cid, pl.ds(sid * ROWS_PT, ROWS_PT)])

    return k(recv_idx3d, edge_feature, zeros)


BN = 1000


def _tc_mlp(node, parts, W1, b1, W2, b2, W3, b3, ln_g, ln_b):
    def body(node_ref, p_ref, w1_ref, b1_ref, w2_ref, b2_ref, w3_ref, b3_ref,
             g_ref, beta_ref, out_ref):
        x = node_ref[...]
        agg = p_ref[0] + p_ref[1]
        h = jnp.dot(x, w1_ref[:D, :], preferred_element_type=jnp.float32)
        h = h + jnp.dot(agg, w1_ref[D:, :], preferred_element_type=jnp.float32)
        h = jax.nn.relu(h + b1_ref[...])
        h = jax.nn.relu(
            jnp.dot(h, w2_ref[...], preferred_element_type=jnp.float32) + b2_ref[...])
        h = jnp.dot(h, w3_ref[...], preferred_element_type=jnp.float32) + b3_ref[...]
        mu = jnp.mean(h, axis=-1, keepdims=True)
        var = jnp.mean((h - mu) * (h - mu), axis=-1, keepdims=True)
        h = (h - mu) * lax.rsqrt(var + 1e-5) * g_ref[...] + beta_ref[...]
        out_ref[...] = h + x

    vec = lambda: pl.BlockSpec((1, D), lambda i: (0, 0))
    return pl.pallas_call(
        body,
        grid=(N // BN,),
        in_specs=[
            pl.BlockSpec((BN, D), lambda i: (i, 0)),
            pl.BlockSpec((NC, BN, D), lambda i: (0, i, 0)),
            pl.BlockSpec((2 * D, D), lambda i: (0, 0)),
            vec(),
            pl.BlockSpec((D, D), lambda i: (0, 0)),
            vec(),
            pl.BlockSpec((D, D), lambda i: (0, 0)),
            vec(),
            vec(),
            vec(),
        ],
        out_specs=pl.BlockSpec((BN, D), lambda i: (i, 0)),
        out_shape=jax.ShapeDtypeStruct((N, D), jnp.float32),
    )(node, parts, W1, b1.reshape(1, D), W2, b2.reshape(1, D), W3,
      b3.reshape(1, D), ln_g.reshape(1, D), ln_b.reshape(1, D))


def kernel(node_feature, edge_matrix, edge_feature, W1, b1, W2, b2, W3, b3,
           ln_g, ln_b):
    recv_idx3d = edge_matrix[1].reshape(NW, NCHUNK, SUB)
    zeros = jnp.zeros((N_PAD, D), dtype=jnp.float32)
    parts = _sc_segment_sum(recv_idx3d, edge_feature, zeros)
    return _tc_mlp(node_feature, parts, W1, b1, W2, b2, W3, b3, ln_g, ln_b)
